# Initial kernel scaffold; baseline (speedup 1.0000x reference)
import jax
import jax.numpy as jnp
from jax.experimental import pallas as pl
from jax.experimental.pallas import tpu as pltpu

_DELTA_VAR = 0.5
_DELTA_DIST = 1.5
_ALPHA = 1.0
_BETA = 1.0
_GAMMA = 0.1
_MAX_ID = 8


def _body(emb_ref, mask_ref, out_ref, acc_ref):
    b = pl.program_id(0)
    nb = pl.num_programs(0)

    e0 = emb_ref[0, 0]
    e1 = emb_ref[0, 1]
    e2 = emb_ref[0, 2]
    e3 = emb_ref[0, 3]
    m = mask_ref[0]

    normsq = e0 * e0 + e1 * e1 + e2 * e2 + e3 * e3
    norm = jnp.sqrt(normsq)

    validf = (m != 0).astype(jnp.float32)
    n_valid = jnp.sum(validf)
    reg_sum = jnp.sum(norm * validf)

    sels = []
    cnts = []
    means = []
    for u in range(1, _MAX_ID):
        sel = m == u
        self_f = sel.astype(jnp.float32)
        cnt = jnp.sum(self_f)
        safe = jnp.maximum(cnt, 1.0)
        mu = (
            jnp.sum(jnp.where(sel, e0, 0.0)) / safe,
            jnp.sum(jnp.where(sel, e1, 0.0)) / safe,
            jnp.sum(jnp.where(sel, e2, 0.0)) / safe,
            jnp.sum(jnp.where(sel, e3, 0.0)) / safe,
        )
        sels.append(sel)
        cnts.append(cnt)
        means.append(mu)

    mc = [jnp.zeros_like(e0) for _ in range(4)]
    for u in range(1, _MAX_ID):
        sel = sels[u - 1]
        mu = means[u - 1]
        for c in range(4):
            mc[c] = jnp.where(sel, mu[c], mc[c])
    d0 = e0 - mc[0]
    d1 = e1 - mc[1]
    d2 = e2 - mc[2]
    d3 = e3 - mc[3]
    dist = jnp.sqrt(d0 * d0 + d1 * d1 + d2 * d2 + d3 * d3)

    num_instances = jnp.float32(0.0)
    var_sum = jnp.float32(0.0)
    for u in range(1, _MAX_ID):
        cnt = cnts[u - 1]
        present = cnt > 0.0
        t = jnp.sum(jnp.where(sels[u - 1], dist, 0.0))
        mean_norm = t / jnp.maximum(cnt, 1.0)
        term = jnp.maximum(mean_norm - _DELTA_VAR, 0.0) ** 2
        var_sum = var_sum + jnp.where(present, term, 0.0)
        num_instances = num_instances + present.astype(jnp.float32)
    var_loss = var_sum / jnp.maximum(num_instances, 1.0)

    dist_sum = jnp.float32(0.0)
    for u in range(_MAX_ID - 1):
        for v in range(u + 1, _MAX_ID - 1):
            mu = means[u]
            mv = means[v]
            pairsq = (
                (mu[0] - mv[0]) ** 2
                + (mu[1] - mv[1]) ** 2
                + (mu[2] - mv[2]) ** 2
                + (mu[3] - mv[3]) ** 2
            )
            pd = jnp.sqrt(pairsq)
            hinge = jnp.maximum(_DELTA_DIST - pd, 0.0) ** 2
            both = jnp.logical_and(cnts[u] > 0.0, cnts[v] > 0.0)
            dist_sum = dist_sum + 2.0 * jnp.where(both, hinge, 0.0)
    denom = num_instances * (num_instances - 1.0)
    dist_loss = jnp.where(
        num_instances > 1.0, dist_sum / jnp.maximum(denom, 1.0), 0.0
    )

    reg_loss = reg_sum / jnp.maximum(n_valid, 1.0)
    loss_b = _ALPHA * var_loss + _BETA * dist_loss + _GAMMA * reg_loss
    inc = (n_valid > 0.0).astype(jnp.float32)

    @pl.when(b == 0)
    def _init():
        acc_ref[0] = 0.0
        acc_ref[1] = 0.0

    acc_ref[0] += loss_b * inc
    acc_ref[1] += inc

    @pl.when(b == nb - 1)
    def _fin():
        s = acc_ref[0]
        n = acc_ref[1]
        out_ref[0, 0] = jnp.where(n > 0.0, s / jnp.maximum(n, 1.0), 0.0)


def kernel(embeddings, instance_mask):
    B, C, H, W = embeddings.shape
    out = pl.pallas_call(
        _body,
        grid=(B,),
        in_specs=[
            pl.BlockSpec((1, C, H, W), lambda b: (b, 0, 0, 0)),
            pl.BlockSpec((1, H, W), lambda b: (b, 0, 0)),
        ],
        out_specs=pl.BlockSpec((1, 1), lambda b: (0, 0)),
        out_shape=jax.ShapeDtypeStruct((1, 1), jnp.float32),
        scratch_shapes=[pltpu.SMEM((2,), jnp.float32)],
    )(embeddings, instance_mask)
    return out[0, 0]


# speedup vs baseline: 6.0675x; 6.0675x over previous
import jax
import jax.numpy as jnp
from jax.experimental import pallas as pl
from jax.experimental.pallas import tpu as pltpu

_DELTA_VAR = 0.5
_DELTA_DIST = 1.5
_ALPHA = 1.0
_BETA = 1.0
_GAMMA = 0.1
_MAX_ID = 8


def _body(emb_ref, mask_ref, out_ref, acc_ref):
    b = pl.program_id(0)
    nb = pl.num_programs(0)

    e0 = emb_ref[0, 0]
    e1 = emb_ref[0, 1]
    e2 = emb_ref[0, 2]
    e3 = emb_ref[0, 3]
    m = mask_ref[0]

    normsq = e0 * e0 + e1 * e1 + e2 * e2 + e3 * e3
    norm = jnp.sqrt(normsq)

    validf = (m != 0).astype(jnp.float32)
    n_valid = jnp.sum(validf)
    reg_sum = jnp.sum(norm * validf)

    sels = []
    cnts = []
    means = []
    for u in range(1, _MAX_ID):
        sel = m == u
        self_f = sel.astype(jnp.float32)
        cnt = jnp.sum(self_f)
        safe = jnp.maximum(cnt, 1.0)
        mu = (
            jnp.sum(jnp.where(sel, e0, 0.0)) / safe,
            jnp.sum(jnp.where(sel, e1, 0.0)) / safe,
            jnp.sum(jnp.where(sel, e2, 0.0)) / safe,
            jnp.sum(jnp.where(sel, e3, 0.0)) / safe,
        )
        sels.append(sel)
        cnts.append(cnt)
        means.append(mu)

    mc = [jnp.zeros_like(e0) for _ in range(4)]
    for u in range(1, _MAX_ID):
        sel = sels[u - 1]
        mu = means[u - 1]
        for c in range(4):
            mc[c] = jnp.where(sel, mu[c], mc[c])
    d0 = e0 - mc[0]
    d1 = e1 - mc[1]
    d2 = e2 - mc[2]
    d3 = e3 - mc[3]
    dist = jnp.sqrt(d0 * d0 + d1 * d1 + d2 * d2 + d3 * d3)

    num_instances = jnp.float32(0.0)
    var_sum = jnp.float32(0.0)
    for u in range(1, _MAX_ID):
        cnt = cnts[u - 1]
        present = cnt > 0.0
        t = jnp.sum(jnp.where(sels[u - 1], dist, 0.0))
        mean_norm = t / jnp.maximum(cnt, 1.0)
        term = jnp.maximum(mean_norm - _DELTA_VAR, 0.0) ** 2
        var_sum = var_sum + jnp.where(present, term, 0.0)
        num_instances = num_instances + present.astype(jnp.float32)
    var_loss = var_sum / jnp.maximum(num_instances, 1.0)

    diag_term = jnp.maximum(jnp.float32(_DELTA_DIST) + jnp.float32(1e6), 0.0) ** 2
    dist_sum = jnp.float32(0.0)
    for u in range(_MAX_ID - 1):
        dist_sum = dist_sum + jnp.where(cnts[u] > 0.0, diag_term, 0.0)
    for u in range(_MAX_ID - 1):
        for v in range(u + 1, _MAX_ID - 1):
            mu = means[u]
            mv = means[v]
            pairsq = (
                (mu[0] - mv[0]) ** 2
                + (mu[1] - mv[1]) ** 2
                + (mu[2] - mv[2]) ** 2
                + (mu[3] - mv[3]) ** 2
            )
            pd = jnp.sqrt(pairsq)
            hinge = jnp.maximum(_DELTA_DIST - pd, 0.0) ** 2
            both = jnp.logical_and(cnts[u] > 0.0, cnts[v] > 0.0)
            dist_sum = dist_sum + 2.0 * jnp.where(both, hinge, 0.0)
    denom = num_instances * (num_instances - 1.0)
    dist_loss = jnp.where(
        num_instances > 1.0, dist_sum / jnp.maximum(denom, 1.0), 0.0
    )

    reg_loss = reg_sum / jnp.maximum(n_valid, 1.0)
    loss_b = _ALPHA * var_loss + _BETA * dist_loss + _GAMMA * reg_loss
    inc = (n_valid > 0.0).astype(jnp.float32)

    @pl.when(b == 0)
    def _init():
        acc_ref[0] = 0.0
        acc_ref[1] = 0.0

    acc_ref[0] += loss_b * inc
    acc_ref[1] += inc

    @pl.when(b == nb - 1)
    def _fin():
        s = acc_ref[0]
        n = acc_ref[1]
        total = jnp.where(n > 0.0, s / jnp.maximum(n, 1.0), 0.0)
        out_ref[:, :] = jnp.broadcast_to(total, (1, 1))


def kernel(embeddings, instance_mask):
    B, C, H, W = embeddings.shape
    out = pl.pallas_call(
        _body,
        grid=(B,),
        in_specs=[
            pl.BlockSpec((1, C, H, W), lambda b: (b, 0, 0, 0)),
            pl.BlockSpec((1, H, W), lambda b: (b, 0, 0)),
        ],
        out_specs=pl.BlockSpec((1, 1), lambda b: (0, 0)),
        out_shape=jax.ShapeDtypeStruct((1, 1), jnp.float32),
        scratch_shapes=[pltpu.SMEM((2,), jnp.float32)],
    )(embeddings, instance_mask)
    return out[0, 0]
